# Initial kernel scaffold; baseline (speedup 1.0000x reference)
import jax
import jax.numpy as jnp
from jax.experimental import pallas as pl


def kernel(x, expert_bias, gate_w):
    raise NotImplementedError("write your pallas kernel here")


# speedup vs baseline: 1.4672x; 1.4672x over previous
import functools

import jax
import jax.numpy as jnp
from jax import lax
from jax.experimental import pallas as pl

NUM_EXPERTS = 64
TOP_K = 8
TOKEN_BLOCK = 512


def _router_body(x_ref, w_ref, b_ref, top_ref, idx_ref, hist_ref):
    i = pl.program_id(0)
    scores = jax.nn.sigmoid(
        jnp.dot(x_ref[...], w_ref[...], preferred_element_type=jnp.float32)
    )
    biased = scores + b_ref[...]
    t = scores.shape[0]
    iota = lax.broadcasted_iota(jnp.int32, (t, NUM_EXPERTS), 1)
    work = biased
    sel = jnp.zeros((t, NUM_EXPERTS), jnp.bool_)
    tops, idxs = [], []
    for _ in range(TOP_K):
        m = jnp.max(work, axis=1, keepdims=True)
        eq = work == m
        idxk = jnp.min(jnp.where(eq, iota, NUM_EXPERTS), axis=1, keepdims=True)
        chosen = iota == idxk
        tops.append(jnp.sum(jnp.where(chosen, scores, 0.0), axis=1, keepdims=True))
        idxs.append(idxk)
        sel = jnp.logical_or(sel, chosen)
        work = jnp.where(chosen, -jnp.inf, work)
    top = jnp.concatenate(tops, axis=1)
    top = top / (jnp.sum(top, axis=1, keepdims=True) + 1e-20)
    top_ref[...] = top
    idx_ref[...] = jnp.concatenate(idxs, axis=1)
    hblk = jnp.sum(sel.astype(jnp.float32), axis=0, keepdims=True)

    @pl.when(i == 0)
    def _init():
        hist_ref[...] = jnp.zeros_like(hist_ref)

    hist_ref[...] += hblk


@jax.jit
def kernel(x, expert_bias, gate_w):
    tokens, dim = x.shape
    e = gate_w.shape[0]
    w_t = gate_w.T
    bias2 = expert_bias.reshape(1, e)
    t = TOKEN_BLOCK
    top, idx, hist = pl.pallas_call(
        _router_body,
        grid=(tokens // t,),
        in_specs=[
            pl.BlockSpec((t, dim), lambda i: (i, 0)),
            pl.BlockSpec((dim, e), lambda i: (0, 0)),
            pl.BlockSpec((1, e), lambda i: (0, 0)),
        ],
        out_specs=[
            pl.BlockSpec((t, TOP_K), lambda i: (i, 0)),
            pl.BlockSpec((t, TOP_K), lambda i: (i, 0)),
            pl.BlockSpec((1, e), lambda i: (0, 0)),
        ],
        out_shape=[
            jax.ShapeDtypeStruct((tokens, TOP_K), jnp.float32),
            jax.ShapeDtypeStruct((tokens, TOP_K), jnp.int32),
            jax.ShapeDtypeStruct((1, e), jnp.float32),
        ],
    )(x, w_t, bias2)
    return top, idx, hist.reshape(e)
